# baseline (speedup 1.0000x reference)
import numpy as np

import jax
import jax.numpy as jnp
from jax import lax
from jax.experimental import pallas as pl
from jax.experimental.pallas import tpu as pltpu

_G = 32

_PERM = np.zeros((64, 134), dtype=np.int32)
for _o in range(64):
    _PERM[_o, 0:66] = _o * 66 + np.arange(66)
    _PERM[_o, 66:130] = 4288 + _o * 64 + np.arange(64)
    _PERM[_o, 130] = 4224 + _o
    _PERM[_o, 131] = 8384 + _o
    _PERM[_o, 132] = 8448 + _o
    _PERM[_o, 133] = 8512


def _tbody(in_ref, out_ref):
    for j in range(67):
        out_ref[0, :, j * 128:(j + 1) * 128] = (
            in_ref[0, j * 128:(j + 1) * 128, :].T)


def _body(s_ref, pv_ref, fine_ref, *rest):
    n_w = 4 * _G
    w_refs = rest[0:n_w]
    out_ref = rest[n_w]
    i = pl.program_id(0)
    C, HW, K = 64, 1024, 128
    io = lax.broadcasted_iota(jnp.int32, (HW, K), 0).astype(jnp.float32)
    for g in range(_G):
        wq = [pv_ref[g, 10:11, c:c + 1] for c in range(4)]
        wp = sum(wq[c] * w_refs[g * 4 + c][0] for c in range(4))
        w1 = wp[:, 0:66]
        w2 = wp[:, 66:130]
        b1 = wp[:, 130:131]
        b2 = wp[:, 131:132]
        w3 = wp[:, 132:133]
        b3 = wp[0:1, 133:134]

        pvg = pv_ref[g]
        b = s_ref[i * _G + g, 4]
        f = fine_ref[b]

        a = jnp.where(io == pvg[0:1, :], pvg[4:5, :], 0.0)
        a = jnp.where(io == pvg[1:2, :], pvg[5:6, :], a)
        a = jnp.where(io == pvg[2:3, :], pvg[6:7, :], a)
        a = jnp.where(io == pvg[3:4, :], pvg[7:8, :], a)
        vf = jnp.dot(f, a, preferred_element_type=jnp.float32)

        cpx = pvg[8:9, :]
        cpy = pvg[9:10, :]
        cpx = (cpx - jnp.min(cpx)) * 4.0
        cpy = (cpy - jnp.min(cpy)) * 4.0
        xcat = jnp.concatenate([vf, cpx, cpy], axis=0)

        h = jnp.dot(w1, xcat, preferred_element_type=jnp.float32)
        h = jnp.maximum(h + b1, 0.0)
        h = jnp.dot(w2, h, preferred_element_type=jnp.float32)
        h = jnp.maximum(h + b2, 0.0)
        out = jnp.sum(h * w3, axis=0, keepdims=True) + b3
        out_ref[g] = out


def kernel(fine_grained_feature, bound_control, py, ct, batch_ind):
    B, C, H, W = fine_grained_feature.shape
    N, K, _ = py.shape
    P = bound_control.shape[1]
    ro = 4.0
    npix = B * H * W

    vpb = bound_control.reshape(B, P, H * W)
    combp = jnp.take(vpb, jnp.asarray(_PERM.reshape(-1)), axis=1)
    combt = pl.pallas_call(
        _tbody,
        grid=(B, (H * W) // 128),
        in_specs=[pl.BlockSpec((1, 64 * 134, 128), lambda b, j: (b, 0, j))],
        out_specs=pl.BlockSpec((1, 128, 64 * 134), lambda b, j: (b, j, 0)),
        out_shape=jax.ShapeDtypeStruct((B, H * W, 64 * 134), jnp.float32),
    )(combp)
    comb = combt.reshape(npix, 64, 134)

    pyp = py / ro
    xs = pyp[..., 0] - 0.5
    ys = pyp[..., 1] - 0.5
    x0 = jnp.floor(xs)
    y0 = jnp.floor(ys)
    fx = xs - x0
    fy = ys - y0

    def _vw(i0, lim):
        v0 = ((i0 >= 0) & (i0 <= lim - 1)).astype(jnp.float32)
        v1 = ((i0 + 1 >= 0) & (i0 + 1 <= lim - 1)).astype(jnp.float32)
        return v0, v1

    vx0, vx1 = _vw(x0, W)
    vy0, vy1 = _vw(y0, H)
    wx0 = (1.0 - fx) * vx0
    wx1 = fx * vx1
    wy0 = (1.0 - fy) * vy0
    wy1 = fy * vy1
    x0c = jnp.clip(x0, 0, W - 1)
    x1c = jnp.clip(x0 + 1, 0, W - 1)
    y0c = jnp.clip(y0, 0, H - 1)
    y1c = jnp.clip(y0 + 1, 0, H - 1)
    sent = float(H * W)
    p00 = jnp.where(vx0 * vy0 > 0, y0c * W + x0c, sent)
    p10 = jnp.where(vx1 * vy0 > 0, y0c * W + x1c, sent)
    p01 = jnp.where(vx0 * vy1 > 0, y1c * W + x0c, sent)
    p11 = jnp.where(vx1 * vy1 > 0, y1c * W + x1c, sent)

    cx = ct[:, 0] - 0.5
    cy = ct[:, 1] - 0.5
    cx0 = jnp.floor(cx)
    cy0 = jnp.floor(cy)
    fcx = cx - cx0
    fcy = cy - cy0
    cvx0, cvx1 = _vw(cx0, W)
    cvy0, cvy1 = _vw(cy0, H)
    wcx0 = (1.0 - fcx) * cvx0
    wcx1 = fcx * cvx1
    wcy0 = (1.0 - fcy) * cvy0
    wcy1 = fcy * cvy1
    wq = jnp.stack([wcx0 * wcy0, wcx1 * wcy0, wcx0 * wcy1, wcx1 * wcy1], -1)
    cx0c = jnp.clip(cx0, 0, W - 1).astype(jnp.int32)
    cx1c = jnp.clip(cx0 + 1, 0, W - 1).astype(jnp.int32)
    cy0c = jnp.clip(cy0, 0, H - 1).astype(jnp.int32)
    cy1c = jnp.clip(cy0 + 1, 0, H - 1).astype(jnp.int32)
    bi = batch_ind.astype(jnp.int32)
    base = bi * (H * W)
    sidx = jnp.stack(
        [base + cy0c * W + cx0c, base + cy0c * W + cx1c,
         base + cy1c * W + cx0c, base + cy1c * W + cx1c, bi], axis=-1)

    wqp = jnp.pad(wq, ((0, 0), (0, K - 4)))
    pv = jnp.stack(
        [p00, p10, p01, p11, wx0 * wy0, wx1 * wy0, wx0 * wy1, wx1 * wy1,
         pyp[..., 0], pyp[..., 1], wqp], axis=1)

    fine3 = fine_grained_feature.reshape(B, C, H * W)

    def _wspec(g, c):
        return pl.BlockSpec(
            (1, 64, 134),
            lambda i, s, g=g, c=c: (s[i * _G + g, c], 0, 0))

    w_specs = [_wspec(g, c) for g in range(_G) for c in range(4)]

    grid_spec = pltpu.PrefetchScalarGridSpec(
        num_scalar_prefetch=1,
        grid=(N // _G,),
        in_specs=[
            pl.BlockSpec((_G, 11, K), lambda i, s: (i, 0, 0)),
            pl.BlockSpec((B, C, H * W), lambda i, s: (0, 0, 0)),
            *w_specs,
        ],
        out_specs=pl.BlockSpec((_G, 1, K), lambda i, s: (i, 0, 0)),
    )

    out = pl.pallas_call(
        _body,
        grid_spec=grid_spec,
        out_shape=jax.ShapeDtypeStruct((N, 1, K), jnp.float32),
    )(sidx, pv, fine3, *([comb] * (4 * _G)))
    return out.reshape(N, K)


# speedup vs baseline: 1.6175x; 1.6175x over previous
import numpy as np

import jax
import jax.numpy as jnp
from jax import lax
from jax.experimental import pallas as pl
from jax.experimental.pallas import tpu as pltpu

_G = 32

_PERM = np.zeros((64, 134), dtype=np.int32)
for _o in range(64):
    _PERM[_o, 0:66] = _o * 66 + np.arange(66)
    _PERM[_o, 66:130] = 4288 + _o * 64 + np.arange(64)
    _PERM[_o, 130] = 4224 + _o
    _PERM[_o, 131] = 8384 + _o
    _PERM[_o, 132] = 8448 + _o
    _PERM[_o, 133] = 8512


def _body(s_ref, pv_ref, fine_ref, *rest):
    n_w = 4 * _G
    w_refs = rest[0:n_w]
    out_ref = rest[n_w]
    i = pl.program_id(0)
    C, HW, K = 64, 1024, 128
    ioc = lax.broadcasted_iota(jnp.int32, (32, K), 0).astype(jnp.float32)
    for g in range(_G):
        wq = [pv_ref[g, 4:5, c:c + 1] for c in range(4)]
        wp = sum(wq[c] * w_refs[g * 4 + c][0] for c in range(4))
        w1 = wp[:, 0:66]
        w2 = wp[:, 66:130]
        b1 = wp[:, 130:131]
        b2 = wp[:, 131:132]
        w3 = wp[:, 132:133]
        b3 = wp[0:1, 133:134]

        pvg = pv_ref[g]
        b = s_ref[i * _G + g, 4]
        f = fine_ref[b]

        tx = jnp.maximum(1.0 - jnp.abs(ioc - pvg[0:1, :]), 0.0)
        ty = jnp.maximum(1.0 - jnp.abs(ioc - pvg[1:2, :]), 0.0)
        a = (ty[:, None, :] * tx[None, :, :]).reshape(HW, K)
        vf = jnp.dot(f, a, preferred_element_type=jnp.float32)

        cpx = pvg[2:3, :]
        cpy = pvg[3:4, :]
        cpx = (cpx - jnp.min(cpx)) * 4.0
        cpy = (cpy - jnp.min(cpy)) * 4.0
        xcat = jnp.concatenate([vf, cpx, cpy], axis=0)

        h = jnp.dot(w1, xcat, preferred_element_type=jnp.float32)
        h = jnp.maximum(h + b1, 0.0)
        h = jnp.dot(w2, h, preferred_element_type=jnp.float32)
        h = jnp.maximum(h + b2, 0.0)
        out = jnp.sum(h * w3, axis=0, keepdims=True) + b3
        out_ref[g] = out


def kernel(fine_grained_feature, bound_control, py, ct, batch_ind):
    B, C, H, W = fine_grained_feature.shape
    N, K, _ = py.shape
    P = bound_control.shape[1]
    ro = 4.0
    npix = B * H * W

    vpb = bound_control.reshape(B, P, H * W)
    comb = jnp.take(vpb, jnp.asarray(_PERM.reshape(-1)), axis=1)
    comb = comb.reshape(B, 64, 134, H * W)
    comb = jnp.transpose(comb, (0, 3, 1, 2)).reshape(npix, 64, 134)

    pyp = py / ro
    xs = pyp[..., 0] - 0.5
    ys = pyp[..., 1] - 0.5

    def _vw(i0, lim):
        v0 = ((i0 >= 0) & (i0 <= lim - 1)).astype(jnp.float32)
        v1 = ((i0 + 1 >= 0) & (i0 + 1 <= lim - 1)).astype(jnp.float32)
        return v0, v1

    cx = ct[:, 0] - 0.5
    cy = ct[:, 1] - 0.5
    cx0 = jnp.floor(cx)
    cy0 = jnp.floor(cy)
    fcx = cx - cx0
    fcy = cy - cy0
    cvx0, cvx1 = _vw(cx0, W)
    cvy0, cvy1 = _vw(cy0, H)
    wcx0 = (1.0 - fcx) * cvx0
    wcx1 = fcx * cvx1
    wcy0 = (1.0 - fcy) * cvy0
    wcy1 = fcy * cvy1
    wq = jnp.stack([wcx0 * wcy0, wcx1 * wcy0, wcx0 * wcy1, wcx1 * wcy1], -1)
    cx0c = jnp.clip(cx0, 0, W - 1).astype(jnp.int32)
    cx1c = jnp.clip(cx0 + 1, 0, W - 1).astype(jnp.int32)
    cy0c = jnp.clip(cy0, 0, H - 1).astype(jnp.int32)
    cy1c = jnp.clip(cy0 + 1, 0, H - 1).astype(jnp.int32)
    bi = batch_ind.astype(jnp.int32)
    base = bi * (H * W)
    sidx = jnp.stack(
        [base + cy0c * W + cx0c, base + cy0c * W + cx1c,
         base + cy1c * W + cx0c, base + cy1c * W + cx1c, bi], axis=-1)

    wqp = jnp.pad(wq, ((0, 0), (0, K - 4)))
    pv = jnp.stack(
        [xs, ys, pyp[..., 0], pyp[..., 1], wqp], axis=1)

    fine3 = fine_grained_feature.reshape(B, C, H * W)

    def _wspec(g, c):
        return pl.BlockSpec(
            (1, 64, 134),
            lambda i, s, g=g, c=c: (s[i * _G + g, c], 0, 0))

    w_specs = [_wspec(g, c) for g in range(_G) for c in range(4)]

    grid_spec = pltpu.PrefetchScalarGridSpec(
        num_scalar_prefetch=1,
        grid=(N // _G,),
        in_specs=[
            pl.BlockSpec((_G, 5, K), lambda i, s: (i, 0, 0)),
            pl.BlockSpec((B, C, H * W), lambda i, s: (0, 0, 0)),
            *w_specs,
        ],
        out_specs=pl.BlockSpec((_G, 1, K), lambda i, s: (i, 0, 0)),
    )

    out = pl.pallas_call(
        _body,
        grid_spec=grid_spec,
        out_shape=jax.ShapeDtypeStruct((N, 1, K), jnp.float32),
    )(sidx, pv, fine3, *([comb] * (4 * _G)))
    return out.reshape(N, K)


# speedup vs baseline: 1.6251x; 1.0047x over previous
import numpy as np

import jax
import jax.numpy as jnp
from jax import lax
from jax.experimental import pallas as pl
from jax.experimental.pallas import tpu as pltpu

_G = 64

_PERM = np.zeros((64, 134), dtype=np.int32)
for _o in range(64):
    _PERM[_o, 0:66] = _o * 66 + np.arange(66)
    _PERM[_o, 66:130] = 4288 + _o * 64 + np.arange(64)
    _PERM[_o, 130] = 4224 + _o
    _PERM[_o, 131] = 8384 + _o
    _PERM[_o, 132] = 8448 + _o
    _PERM[_o, 133] = 8512


def _body(s_ref, pv_ref, fine_ref, *rest):
    n_w = 4 * _G
    w_refs = rest[0:n_w]
    out_ref = rest[n_w]
    i = pl.program_id(0)
    C, HW, K = 64, 1024, 128
    ioc = lax.broadcasted_iota(jnp.int32, (32, K), 0).astype(jnp.float32)
    for g in range(_G):
        wq = [pv_ref[g, 4:5, c:c + 1] for c in range(4)]
        wp = sum(wq[c] * w_refs[g * 4 + c][0] for c in range(4))
        w1 = wp[:, 0:66]
        w2 = wp[:, 66:130]
        b1 = wp[:, 130:131]
        b2 = wp[:, 131:132]
        w3 = wp[:, 132:133]
        b3 = wp[0:1, 133:134]

        pvg = pv_ref[g]
        b = s_ref[i * _G + g, 4]
        f = fine_ref[b]

        tx = jnp.maximum(1.0 - jnp.abs(ioc - pvg[0:1, :]), 0.0)
        ty = jnp.maximum(1.0 - jnp.abs(ioc - pvg[1:2, :]), 0.0)
        a = (ty[:, None, :] * tx[None, :, :]).reshape(HW, K)
        vf = jnp.dot(f, a, preferred_element_type=jnp.float32)

        cpx = pvg[2:3, :]
        cpy = pvg[3:4, :]
        cpx = (cpx - jnp.min(cpx)) * 4.0
        cpy = (cpy - jnp.min(cpy)) * 4.0
        xcat = jnp.concatenate([vf, cpx, cpy], axis=0)

        h = jnp.dot(w1, xcat, preferred_element_type=jnp.float32)
        h = jnp.maximum(h + b1, 0.0)
        h = jnp.dot(w2, h, preferred_element_type=jnp.float32)
        h = jnp.maximum(h + b2, 0.0)
        out = jnp.sum(h * w3, axis=0, keepdims=True) + b3
        out_ref[g] = out


def kernel(fine_grained_feature, bound_control, py, ct, batch_ind):
    B, C, H, W = fine_grained_feature.shape
    N, K, _ = py.shape
    P = bound_control.shape[1]
    ro = 4.0
    npix = B * H * W

    vpb = bound_control.reshape(B, P, H * W)
    comb = jnp.take(vpb, jnp.asarray(_PERM.reshape(-1)), axis=1)
    comb = comb.reshape(B, 64, 134, H * W)
    comb = jnp.transpose(comb, (0, 3, 1, 2)).reshape(npix, 64, 134)

    pyp = py / ro
    xs = pyp[..., 0] - 0.5
    ys = pyp[..., 1] - 0.5

    def _vw(i0, lim):
        v0 = ((i0 >= 0) & (i0 <= lim - 1)).astype(jnp.float32)
        v1 = ((i0 + 1 >= 0) & (i0 + 1 <= lim - 1)).astype(jnp.float32)
        return v0, v1

    cx = ct[:, 0] - 0.5
    cy = ct[:, 1] - 0.5
    cx0 = jnp.floor(cx)
    cy0 = jnp.floor(cy)
    fcx = cx - cx0
    fcy = cy - cy0
    cvx0, cvx1 = _vw(cx0, W)
    cvy0, cvy1 = _vw(cy0, H)
    wcx0 = (1.0 - fcx) * cvx0
    wcx1 = fcx * cvx1
    wcy0 = (1.0 - fcy) * cvy0
    wcy1 = fcy * cvy1
    wq = jnp.stack([wcx0 * wcy0, wcx1 * wcy0, wcx0 * wcy1, wcx1 * wcy1], -1)
    cx0c = jnp.clip(cx0, 0, W - 1).astype(jnp.int32)
    cx1c = jnp.clip(cx0 + 1, 0, W - 1).astype(jnp.int32)
    cy0c = jnp.clip(cy0, 0, H - 1).astype(jnp.int32)
    cy1c = jnp.clip(cy0 + 1, 0, H - 1).astype(jnp.int32)
    bi = batch_ind.astype(jnp.int32)
    base = bi * (H * W)
    sidx = jnp.stack(
        [base + cy0c * W + cx0c, base + cy0c * W + cx1c,
         base + cy1c * W + cx0c, base + cy1c * W + cx1c, bi], axis=-1)

    wqp = jnp.pad(wq, ((0, 0), (0, K - 4)))
    pv = jnp.stack(
        [xs, ys, pyp[..., 0], pyp[..., 1], wqp], axis=1)

    fine3 = fine_grained_feature.reshape(B, C, H * W)

    def _wspec(g, c):
        return pl.BlockSpec(
            (1, 64, 134),
            lambda i, s, g=g, c=c: (s[i * _G + g, c], 0, 0))

    w_specs = [_wspec(g, c) for g in range(_G) for c in range(4)]

    grid_spec = pltpu.PrefetchScalarGridSpec(
        num_scalar_prefetch=1,
        grid=(N // _G,),
        in_specs=[
            pl.BlockSpec((_G, 5, K), lambda i, s: (i, 0, 0)),
            pl.BlockSpec((B, C, H * W), lambda i, s: (0, 0, 0)),
            *w_specs,
        ],
        out_specs=pl.BlockSpec((_G, 1, K), lambda i, s: (i, 0, 0)),
    )

    out = pl.pallas_call(
        _body,
        grid_spec=grid_spec,
        out_shape=jax.ShapeDtypeStruct((N, 1, K), jnp.float32),
    )(sidx, pv, fine3, *([comb] * (4 * _G)))
    return out.reshape(N, K)
